# Initial kernel scaffold; baseline (speedup 1.0000x reference)
import jax
import jax.numpy as jnp
from jax.experimental import pallas as pl


def kernel(x, retrieval_features, W_base, b_base, A_gen_W, A_gen_b, B_experts, router_W, router_b):
    raise NotImplementedError("write your pallas kernel here")


# speedup vs baseline: 2.9315x; 2.9315x over previous
import functools

import jax
import jax.numpy as jnp
from jax.experimental import pallas as pl

_B, _S, _D = 4, 4096, 2048
_R = 512
_RANK = 8
_E = 2
_CHUNK = 128
_ALPHA = 16.0

_TS = 512
_TD = 1024
_NC = _TS // _CHUNK
_ER = _E * _RANK

_KT = 4096


def _a_gen_kernel(rf_ref, w_ref, b_ref, out_ref):
    out_ref[0] = (
        jnp.dot(rf_ref[...], w_ref[0], preferred_element_type=jnp.float32)
        + b_ref[0]
    )


def _main_kernel(x_ref, w_ref, b_ref, a_ref, bex_ref, rw_ref, rb_ref,
                 out_ref, idx_ref):
    xb = x_ref[0]

    row_c = jax.lax.broadcasted_iota(jnp.int32, (_NC, _TS), 0)
    col_t = jax.lax.broadcasted_iota(jnp.int32, (_NC, _TS), 1)
    pool = (col_t // _CHUNK == row_c).astype(jnp.float32)
    cm = jnp.dot(pool, xb, preferred_element_type=jnp.float32) * (1.0 / _CHUNK)
    logits = jnp.dot(cm, rw_ref[...], preferred_element_type=jnp.float32) + rb_ref[...]
    m = jnp.max(logits, axis=-1, keepdims=True)
    ex = jnp.exp(logits - m)
    wc = ex / jnp.sum(ex, axis=-1, keepdims=True)
    idx_ref[0] = (wc[:, 1:2] > wc[:, 0:1]).astype(jnp.int32)

    row_t = jax.lax.broadcasted_iota(jnp.int32, (_TS, _NC), 0)
    col_c = jax.lax.broadcasted_iota(jnp.int32, (_TS, _NC), 1)
    unpool = (row_t // _CHUNK == col_c).astype(jnp.float32)
    w_tok = jnp.dot(unpool, wc, preferred_element_type=jnp.float32)
    row_e = jax.lax.broadcasted_iota(jnp.int32, (_E, _ER), 0)
    col_er = jax.lax.broadcasted_iota(jnp.int32, (_E, _ER), 1)
    expand = (col_er // _RANK == row_e).astype(jnp.float32)
    w_exp = jnp.dot(w_tok, expand, preferred_element_type=jnp.float32)

    h = jnp.dot(xb, a_ref[0], preferred_element_type=jnp.float32)
    hw = h * w_exp * (_ALPHA / _RANK)
    out_ref[0] = (
        jnp.dot(xb, w_ref[...], preferred_element_type=jnp.float32)
        + b_ref[...]
        + jnp.dot(hw, bex_ref[...], preferred_element_type=jnp.float32)
    )


@jax.jit
def kernel(x, retrieval_features, W_base, b_base, A_gen_W, A_gen_b,
           B_experts, router_W, router_b):
    Bsz, S, D = x.shape
    E, R, DR = A_gen_W.shape
    rank = DR // D
    er = E * rank
    n_si = S // _TS
    n_ti = D // _TD
    nk = DR // _KT

    a_flat = pl.pallas_call(
        _a_gen_kernel,
        grid=(E, nk),
        in_specs=[
            pl.BlockSpec((Bsz, R), lambda e, k: (0, 0)),
            pl.BlockSpec((1, R, _KT), lambda e, k: (e, 0, k)),
            pl.BlockSpec((1, 1, _KT), lambda e, k: (e, 0, k)),
        ],
        out_specs=pl.BlockSpec((1, Bsz, _KT), lambda e, k: (e, 0, k)),
        out_shape=jax.ShapeDtypeStruct((E, Bsz, DR), jnp.float32),
    )(retrieval_features, A_gen_W, A_gen_b.reshape(E, 1, DR))

    a_all = a_flat.reshape(E, Bsz, D, rank).transpose(1, 2, 0, 3).reshape(Bsz, D, er)
    bex = B_experts.reshape(er, D)
    bvec = b_base.reshape(1, D)
    rbvec = router_b.reshape(1, E)

    out, idx_c = pl.pallas_call(
        _main_kernel,
        grid=(n_ti, Bsz, n_si),
        in_specs=[
            pl.BlockSpec((1, _TS, D), lambda ti, b, si: (b, si, 0)),
            pl.BlockSpec((D, _TD), lambda ti, b, si: (0, ti)),
            pl.BlockSpec((1, _TD), lambda ti, b, si: (0, ti)),
            pl.BlockSpec((1, D, er), lambda ti, b, si: (b, 0, 0)),
            pl.BlockSpec((er, _TD), lambda ti, b, si: (0, ti)),
            pl.BlockSpec((D, E), lambda ti, b, si: (0, 0)),
            pl.BlockSpec((1, E), lambda ti, b, si: (0, 0)),
        ],
        out_specs=[
            pl.BlockSpec((1, _TS, _TD), lambda ti, b, si: (b, si, ti)),
            pl.BlockSpec((1, _NC, 1), lambda ti, b, si: (b * n_si + si, 0, 0)),
        ],
        out_shape=[
            jax.ShapeDtypeStruct((Bsz, S, D), jnp.float32),
            jax.ShapeDtypeStruct((Bsz * n_si, _NC, 1), jnp.int32),
        ],
    )(x, W_base, bvec, a_all, bex, router_W, rbvec)

    expert_indices = jnp.broadcast_to(
        idx_c.reshape(Bsz, S // _CHUNK, 1), (Bsz, S // _CHUNK, _CHUNK)
    ).reshape(Bsz, S)
    return out, expert_indices


# speedup vs baseline: 3.5128x; 1.1983x over previous
import functools

import jax
import jax.numpy as jnp
from jax.experimental import pallas as pl

_B, _S, _D = 4, 4096, 2048
_R = 512
_RANK = 8
_E = 2
_CHUNK = 128
_ALPHA = 16.0

_TS = 512
_TD = 2048
_NC = _TS // _CHUNK
_ER = _E * _RANK

_KT = 4096


def _a_gen_kernel(rf_ref, w_ref, b_ref, out_ref):
    out_ref[0] = (
        jnp.dot(rf_ref[...], w_ref[0], preferred_element_type=jnp.float32)
        + b_ref[0]
    )


def _main_kernel(x_ref, w_ref, b_ref, a_ref, bex_ref, rw_ref, rb_ref,
                 out_ref, idx_ref):
    xb = x_ref[0]

    row_c = jax.lax.broadcasted_iota(jnp.int32, (_NC, _TS), 0)
    col_t = jax.lax.broadcasted_iota(jnp.int32, (_NC, _TS), 1)
    pool = (col_t // _CHUNK == row_c).astype(jnp.float32)
    cm = jnp.dot(pool, xb, preferred_element_type=jnp.float32) * (1.0 / _CHUNK)
    logits = jnp.dot(cm, rw_ref[...], preferred_element_type=jnp.float32) + rb_ref[...]
    m = jnp.max(logits, axis=-1, keepdims=True)
    ex = jnp.exp(logits - m)
    wc = ex / jnp.sum(ex, axis=-1, keepdims=True)
    idx_ref[0] = (wc[:, 1:2] > wc[:, 0:1]).astype(jnp.int32)

    row_t = jax.lax.broadcasted_iota(jnp.int32, (_TS, _NC), 0)
    col_c = jax.lax.broadcasted_iota(jnp.int32, (_TS, _NC), 1)
    unpool = (row_t // _CHUNK == col_c).astype(jnp.float32)
    w_tok = jnp.dot(unpool, wc, preferred_element_type=jnp.float32)
    row_e = jax.lax.broadcasted_iota(jnp.int32, (_E, _ER), 0)
    col_er = jax.lax.broadcasted_iota(jnp.int32, (_E, _ER), 1)
    expand = (col_er // _RANK == row_e).astype(jnp.float32)
    w_exp = jnp.dot(w_tok, expand, preferred_element_type=jnp.float32)

    h = jnp.dot(xb, a_ref[0], preferred_element_type=jnp.float32)
    hw = h * w_exp * (_ALPHA / _RANK)
    out_ref[0] = (
        jnp.dot(xb, w_ref[...], preferred_element_type=jnp.float32)
        + b_ref[...]
        + jnp.dot(hw, bex_ref[...], preferred_element_type=jnp.float32)
    )


@jax.jit
def kernel(x, retrieval_features, W_base, b_base, A_gen_W, A_gen_b,
           B_experts, router_W, router_b):
    Bsz, S, D = x.shape
    E, R, DR = A_gen_W.shape
    rank = DR // D
    er = E * rank
    n_si = S // _TS
    n_ti = D // _TD
    nk = DR // _KT

    a_flat = pl.pallas_call(
        _a_gen_kernel,
        grid=(E, nk),
        in_specs=[
            pl.BlockSpec((Bsz, R), lambda e, k: (0, 0)),
            pl.BlockSpec((1, R, _KT), lambda e, k: (e, 0, k)),
            pl.BlockSpec((1, 1, _KT), lambda e, k: (e, 0, k)),
        ],
        out_specs=pl.BlockSpec((1, Bsz, _KT), lambda e, k: (e, 0, k)),
        out_shape=jax.ShapeDtypeStruct((E, Bsz, DR), jnp.float32),
    )(retrieval_features, A_gen_W, A_gen_b.reshape(E, 1, DR))

    a_all = a_flat.reshape(E, Bsz, D, rank).transpose(1, 2, 0, 3).reshape(Bsz, D, er)
    bex = B_experts.reshape(er, D)
    bvec = b_base.reshape(1, D)
    rbvec = router_b.reshape(1, E)

    out, idx_c = pl.pallas_call(
        _main_kernel,
        grid=(n_ti, Bsz, n_si),
        in_specs=[
            pl.BlockSpec((1, _TS, D), lambda ti, b, si: (b, si, 0)),
            pl.BlockSpec((D, _TD), lambda ti, b, si: (0, ti)),
            pl.BlockSpec((1, _TD), lambda ti, b, si: (0, ti)),
            pl.BlockSpec((1, D, er), lambda ti, b, si: (b, 0, 0)),
            pl.BlockSpec((er, _TD), lambda ti, b, si: (0, ti)),
            pl.BlockSpec((D, E), lambda ti, b, si: (0, 0)),
            pl.BlockSpec((1, E), lambda ti, b, si: (0, 0)),
        ],
        out_specs=[
            pl.BlockSpec((1, _TS, _TD), lambda ti, b, si: (b, si, ti)),
            pl.BlockSpec((1, _NC, 1), lambda ti, b, si: (b * n_si + si, 0, 0)),
        ],
        out_shape=[
            jax.ShapeDtypeStruct((Bsz, S, D), jnp.float32),
            jax.ShapeDtypeStruct((Bsz * n_si, _NC, 1), jnp.int32),
        ],
    )(x, W_base, bvec, a_all, bex, router_W, rbvec)

    expert_indices = jnp.broadcast_to(
        idx_c.reshape(Bsz, S // _CHUNK, 1), (Bsz, S // _CHUNK, _CHUNK)
    ).reshape(Bsz, S)
    return out, expert_indices
